# Initial kernel scaffold; baseline (speedup 1.0000x reference)
import functools

import jax
import jax.numpy as jnp
from jax import lax
from jax.experimental import pallas as pl
from jax.experimental.pallas import tpu as pltpu
from jax.experimental.pallas import tpu_sc as plsc

NC = 2
NS = 16
NW = NC * NS
L = 16

CHUNK = 80


def _decoder_kernel(z_hbm, src_hbm, dst_hbm, out_hbm,
                    idx_s, idx_d, rows_s, rows_d, out_v, sem_s, sem_d):
    b = out_hbm.shape[0]
    d = z_hbm.shape[1]
    b_per_w = b // NW

    wid = lax.axis_index("s") * NC + lax.axis_index("c")
    base = wid * b_per_w
    n_chunks = b_per_w // CHUNK

    def chunk_body(ci, carry):
        off = base + ci * CHUNK
        pltpu.sync_copy(src_hbm.at[pl.ds(off, CHUNK)], idx_s)
        pltpu.sync_copy(dst_hbm.at[pl.ds(off, CHUNK)], idx_d)
        cp_s = pltpu.async_copy(z_hbm.at[idx_s], rows_s, sem_s)
        cp_d = pltpu.async_copy(z_hbm.at[idx_d], rows_d, sem_d)
        cp_s.wait()
        cp_d.wait()

        def edge_body(e, carry2):
            acc = rows_s[e, pl.ds(0, L)] * rows_d[e, pl.ds(0, L)]
            for k in range(1, d // L):
                acc = acc + rows_s[e, pl.ds(k * L, L)] * rows_d[e, pl.ds(k * L, L)]
            out_v[e] = jnp.sum(acc)
            return carry2

        lax.fori_loop(0, CHUNK, edge_body, 0)
        pltpu.sync_copy(out_v, out_hbm.at[pl.ds(off, CHUNK)])
        return carry

    lax.fori_loop(0, n_chunks, chunk_body, 0)


def kernel(z, source, destination):
    b = source.shape[0]
    d = z.shape[1]
    source = source.astype(jnp.int32)
    destination = destination.astype(jnp.int32)

    run = functools.partial(
        pl.kernel,
        mesh=plsc.VectorSubcoreMesh(core_axis_name="c", subcore_axis_name="s"),
        out_type=jax.ShapeDtypeStruct((b,), jnp.float32),
        scratch_types=[
            pltpu.VMEM((CHUNK,), jnp.int32),
            pltpu.VMEM((CHUNK,), jnp.int32),
            pltpu.VMEM((CHUNK, d), jnp.float32),
            pltpu.VMEM((CHUNK, d), jnp.float32),
            pltpu.VMEM((CHUNK,), jnp.float32),
            pltpu.SemaphoreType.DMA,
            pltpu.SemaphoreType.DMA,
        ],
    )(_decoder_kernel)
    return run(z, source, destination)


# speedup vs baseline: 4.1565x; 4.1565x over previous
import functools

import jax
import jax.numpy as jnp
from jax import lax
from jax.experimental import pallas as pl
from jax.experimental.pallas import tpu as pltpu
from jax.experimental.pallas import tpu_sc as plsc

NC = 2
NS = 16
NW = NC * NS
L = 16

CHUNK = 80


def _decoder_kernel(z_hbm, src_hbm, dst_hbm, out_hbm,
                    idx_s, idx_d, rows_s, rows_d, out_v, sem_s, sem_d):
    b = out_hbm.shape[0]
    d = z_hbm.shape[1]
    b_per_w = b // NW

    wid = lax.axis_index("s") * NC + lax.axis_index("c")
    base = wid * b_per_w
    n_chunks = b_per_w // CHUNK

    def chunk_body(ci, carry):
        off = base + ci * CHUNK
        pltpu.sync_copy(src_hbm.at[pl.ds(off, CHUNK)], idx_s)
        pltpu.sync_copy(dst_hbm.at[pl.ds(off, CHUNK)], idx_d)
        cp_s = pltpu.async_copy(z_hbm.at[idx_s], rows_s, sem_s)
        cp_d = pltpu.async_copy(z_hbm.at[idx_d], rows_d, sem_d)
        cp_s.wait()
        cp_d.wait()

        lane = lax.iota(jnp.int32, L)

        def group_body(g, carry2):
            def edge_body(e, tot):
                ei = g * L + e
                acc = rows_s[ei, pl.ds(0, L)] * rows_d[ei, pl.ds(0, L)]
                for k in range(1, d // L):
                    acc = acc + rows_s[ei, pl.ds(k * L, L)] * rows_d[ei, pl.ds(k * L, L)]
                val = jnp.sum(acc)
                return jnp.where(lane == e, jnp.full((L,), val, jnp.float32), tot)

            tot = lax.fori_loop(0, L, edge_body, jnp.zeros((L,), jnp.float32))
            out_v[pl.ds(g * L, L)] = tot
            return carry2

        lax.fori_loop(0, CHUNK // L, group_body, 0)
        pltpu.sync_copy(out_v, out_hbm.at[pl.ds(off, CHUNK)])
        return carry

    lax.fori_loop(0, n_chunks, chunk_body, 0)


def kernel(z, source, destination):
    b = source.shape[0]
    d = z.shape[1]
    source = source.astype(jnp.int32)
    destination = destination.astype(jnp.int32)

    run = functools.partial(
        pl.kernel,
        mesh=plsc.VectorSubcoreMesh(core_axis_name="c", subcore_axis_name="s"),
        compiler_params=pltpu.CompilerParams(needs_layout_passes=False),
        out_type=jax.ShapeDtypeStruct((b,), jnp.float32),
        scratch_types=[
            pltpu.VMEM((CHUNK,), jnp.int32),
            pltpu.VMEM((CHUNK,), jnp.int32),
            pltpu.VMEM((CHUNK, d), jnp.float32),
            pltpu.VMEM((CHUNK, d), jnp.float32),
            pltpu.VMEM((CHUNK,), jnp.float32),
            pltpu.SemaphoreType.DMA,
            pltpu.SemaphoreType.DMA,
        ],
    )(_decoder_kernel)
    return run(z, source, destination)


# speedup vs baseline: 9.9517x; 2.3943x over previous
import functools

import jax
import jax.numpy as jnp
from jax import lax
from jax.experimental import pallas as pl
from jax.experimental.pallas import tpu as pltpu
from jax.experimental.pallas import tpu_sc as plsc

NC = 2
NS = 16
NW = NC * NS
L = 16

CHUNK = 80


def _decoder_kernel(z_hbm, src_hbm, dst_hbm, out_hbm,
                    idx_s, idx_d, rows_s, rows_d, out_v,
                    sem_is, sem_id, sem_s0, sem_d0, sem_s1, sem_d1):
    b = out_hbm.shape[0]
    d = z_hbm.shape[1]
    b_per_w = b // NW

    wid = lax.axis_index("s") * NC + lax.axis_index("c")
    base = wid * b_per_w
    n_chunks = b_per_w // CHUNK

    cp_is = pltpu.async_copy(src_hbm.at[pl.ds(base, b_per_w)], idx_s, sem_is)
    cp_id = pltpu.async_copy(dst_hbm.at[pl.ds(base, b_per_w)], idx_d, sem_id)
    cp_is.wait()
    cp_id.wait()

    sems = ((sem_s0, sem_d0), (sem_s1, sem_d1))

    def issue(ci, buf):
        ss, sd = sems[buf]
        cs = pltpu.async_copy(
            z_hbm.at[idx_s.at[pl.ds(ci * CHUNK, CHUNK)]],
            rows_s.at[buf], ss)
        cd = pltpu.async_copy(
            z_hbm.at[idx_d.at[pl.ds(ci * CHUNK, CHUNK)]],
            rows_d.at[buf], sd)
        return cs, cd

    def wait(buf):
        ss, sd = sems[buf]
        pltpu.make_async_copy(z_hbm.at[idx_s.at[pl.ds(0, CHUNK)]],
                              rows_s.at[buf], ss).wait()
        pltpu.make_async_copy(z_hbm.at[idx_d.at[pl.ds(0, CHUNK)]],
                              rows_d.at[buf], sd).wait()

    lane = lax.iota(jnp.int32, L)

    def compute(ci, buf):
        rs = rows_s.at[buf]
        rd = rows_d.at[buf]

        def group_wrap(g, carry2):
            def edge_body(e, tot):
                acc = rs[e, pl.ds(0, L)] * rd[e, pl.ds(0, L)]
                for k in range(1, d // L):
                    acc = acc + rs[e, pl.ds(k * L, L)] * rd[e, pl.ds(k * L, L)]
                val = jnp.sum(acc)
                return jnp.where(lane == e - g * L,
                                 jnp.full((L,), val, jnp.float32), tot)

            tot = lax.fori_loop(g * L, (g + 1) * L, edge_body,
                                jnp.zeros((L,), jnp.float32))
            out_v[pl.ds(ci * CHUNK + g * L, L)] = tot
            return carry2

        lax.fori_loop(0, CHUNK // L, group_wrap, 0)

    issue(0, 0)

    def pair_body(h, carry):
        i = h * 2
        issue(i + 1, 1)
        wait(0)
        compute(i, 0)
        issue(i + 2, 0)
        wait(1)
        compute(i + 1, 1)
        return carry

    lax.fori_loop(0, (n_chunks - 1) // 2, pair_body, 0)
    wait(0)
    compute(n_chunks - 1, 0)

    pltpu.sync_copy(out_v, out_hbm.at[pl.ds(base, b_per_w)])


def kernel(z, source, destination):
    b = source.shape[0]
    d = z.shape[1]
    b_per_w = b // NW
    source = source.astype(jnp.int32)
    destination = destination.astype(jnp.int32)

    run = functools.partial(
        pl.kernel,
        mesh=plsc.VectorSubcoreMesh(core_axis_name="c", subcore_axis_name="s"),
        compiler_params=pltpu.CompilerParams(needs_layout_passes=False),
        out_type=jax.ShapeDtypeStruct((b,), jnp.float32),
        scratch_types=[
            pltpu.VMEM((b_per_w,), jnp.int32),
            pltpu.VMEM((b_per_w,), jnp.int32),
            pltpu.VMEM((2, CHUNK, d), jnp.float32),
            pltpu.VMEM((2, CHUNK, d), jnp.float32),
            pltpu.VMEM((b_per_w,), jnp.float32),
            pltpu.SemaphoreType.DMA,
            pltpu.SemaphoreType.DMA,
            pltpu.SemaphoreType.DMA,
            pltpu.SemaphoreType.DMA,
            pltpu.SemaphoreType.DMA,
            pltpu.SemaphoreType.DMA,
        ],
    )(_decoder_kernel)
    return run(z, source, destination)


# speedup vs baseline: 11.7632x; 1.1820x over previous
import functools

import jax
import jax.numpy as jnp
from jax import lax
from jax.experimental import pallas as pl
from jax.experimental.pallas import tpu as pltpu
from jax.experimental.pallas import tpu_sc as plsc

NC = 2
NS = 16
NW = NC * NS
L = 16

CHUNK = 80


def _decoder_kernel(z_hbm, src_hbm, dst_hbm, out_hbm,
                    idx_s, idx_d, rows_s, rows_d, out_v,
                    sem_is, sem_id, sem_s0, sem_d0, sem_s1, sem_d1):
    b = out_hbm.shape[0]
    w = z_hbm.shape[1]
    b_per_w = b // NW

    wid = lax.axis_index("s") * NC + lax.axis_index("c")
    base = wid * b_per_w
    n_chunks = b_per_w // CHUNK

    cp_is = pltpu.async_copy(src_hbm.at[pl.ds(base, b_per_w)], idx_s, sem_is)
    cp_id = pltpu.async_copy(dst_hbm.at[pl.ds(base, b_per_w)], idx_d, sem_id)
    cp_is.wait()
    cp_id.wait()

    sems = ((sem_s0, sem_d0), (sem_s1, sem_d1))

    def issue(ci, buf):
        ss, sd = sems[buf]
        cs = pltpu.async_copy(
            z_hbm.at[idx_s.at[pl.ds(ci * CHUNK, CHUNK)]],
            rows_s.at[buf], ss)
        cd = pltpu.async_copy(
            z_hbm.at[idx_d.at[pl.ds(ci * CHUNK, CHUNK)]],
            rows_d.at[buf], sd)
        return cs, cd

    def wait(buf):
        ss, sd = sems[buf]
        pltpu.make_async_copy(z_hbm.at[idx_s.at[pl.ds(0, CHUNK)]],
                              rows_s.at[buf], ss).wait()
        pltpu.make_async_copy(z_hbm.at[idx_d.at[pl.ds(0, CHUNK)]],
                              rows_d.at[buf], sd).wait()

    lane = lax.iota(jnp.int32, L)

    def compute(ci, buf):
        rs = rows_s.at[buf]
        rd = rows_d.at[buf]

        def group_wrap(g, carry2):
            def edge_body(e, tot):
                acc = jnp.zeros((L,), jnp.float32)
                for k in range(w // L):
                    vs = plsc.bitcast(rs[e, pl.ds(k * L, L)], jnp.bfloat16)
                    vd = plsc.bitcast(rd[e, pl.ds(k * L, L)], jnp.bfloat16)
                    p = vs * vd
                    u0, u1 = plsc.unpack(p, format=plsc.PackFormat.INTERLEAVED)
                    acc = acc + u0 + u1
                val = jnp.sum(acc)
                return jnp.where(lane == e - g * L,
                                 jnp.full((L,), val, jnp.float32), tot)

            tot = lax.fori_loop(g * L, (g + 1) * L, edge_body,
                                jnp.zeros((L,), jnp.float32))
            out_v[pl.ds(ci * CHUNK + g * L, L)] = tot
            return carry2

        lax.fori_loop(0, CHUNK // L, group_wrap, 0)

    issue(0, 0)

    def pair_body(h, carry):
        i = h * 2
        issue(i + 1, 1)
        wait(0)
        compute(i, 0)
        issue(i + 2, 0)
        wait(1)
        compute(i + 1, 1)
        return carry

    lax.fori_loop(0, (n_chunks - 1) // 2, pair_body, 0)
    wait(0)
    compute(n_chunks - 1, 0)

    pltpu.sync_copy(out_v, out_hbm.at[pl.ds(base, b_per_w)])


def kernel(z, source, destination):
    b = source.shape[0]
    d = z.shape[1]
    b_per_w = b // NW
    z = lax.bitcast_convert_type(
        z.astype(jnp.bfloat16).reshape(z.shape[0], d // 2, 2), jnp.int32)
    source = source.astype(jnp.int32)
    destination = destination.astype(jnp.int32)

    run = functools.partial(
        pl.kernel,
        mesh=plsc.VectorSubcoreMesh(core_axis_name="c", subcore_axis_name="s"),
        compiler_params=pltpu.CompilerParams(
            needs_layout_passes=False, use_tc_tiling_on_sc=False),
        out_type=jax.ShapeDtypeStruct((b,), jnp.float32),
        scratch_types=[
            pltpu.VMEM((b_per_w,), jnp.int32),
            pltpu.VMEM((b_per_w,), jnp.int32),
            pltpu.VMEM((2, CHUNK, d // 2), jnp.int32),
            pltpu.VMEM((2, CHUNK, d // 2), jnp.int32),
            pltpu.VMEM((b_per_w,), jnp.float32),
            pltpu.SemaphoreType.DMA,
            pltpu.SemaphoreType.DMA,
            pltpu.SemaphoreType.DMA,
            pltpu.SemaphoreType.DMA,
            pltpu.SemaphoreType.DMA,
            pltpu.SemaphoreType.DMA,
        ],
    )(_decoder_kernel)
    return run(z, source, destination)
